# baseline (speedup 1.0000x reference)
import functools

import jax
import jax.numpy as jnp
from jax import lax
from jax.experimental import pallas as pl
from jax.experimental.pallas import tpu as pltpu
from jax.experimental.pallas import tpu_sc as plsc

_INFO = plsc.get_sparse_core_info()
_NC = _INFO.num_cores
_NS = _INFO.num_subcores
_NW = _NC * _NS


@functools.cache
def _make_copy(n_rows: int, dim: int):
    rows_per_w = n_rows // _NW
    mesh = plsc.VectorSubcoreMesh(core_axis_name="c", subcore_axis_name="s")

    @functools.partial(
        pl.kernel,
        out_type=jax.ShapeDtypeStruct((n_rows, dim), jnp.float32),
        mesh=mesh,
    )
    def copy_kernel(table_hbm, out_hbm):
        wid = lax.axis_index("s") * _NC + lax.axis_index("c")
        base = wid * rows_per_w
        pltpu.sync_copy(
            table_hbm.at[pl.ds(base, rows_per_w)],
            out_hbm.at[pl.ds(base, rows_per_w)],
        )

    return copy_kernel


def kernel(x, position_embeddings):
    n_rows = x.shape[1]
    dim = position_embeddings.shape[1]
    return _make_copy(n_rows, dim)(position_embeddings)


# speedup vs baseline: 24.7199x; 24.7199x over previous
import functools

import jax
import jax.numpy as jnp
from jax import lax
from jax.experimental import pallas as pl
from jax.experimental.pallas import tpu as pltpu
from jax.experimental.pallas import tpu_sc as plsc

_INFO = plsc.get_sparse_core_info()
_NC = _INFO.num_cores
_NS = _INFO.num_subcores
_NW = _NC * _NS

_CHUNK = 32
_NBUF = 3


@functools.cache
def _make_copy(n_rows: int, dim: int):
    rows_per_w = n_rows // _NW
    n_chunks = rows_per_w // _CHUNK
    mesh = plsc.VectorSubcoreMesh(core_axis_name="c", subcore_axis_name="s")

    @functools.partial(
        pl.kernel,
        out_type=jax.ShapeDtypeStruct((n_rows, dim), jnp.float32),
        mesh=mesh,
        scratch_types=[
            pltpu.VMEM((_NBUF, _CHUNK, dim), jnp.float32),
            pltpu.SemaphoreType.DMA,
            pltpu.SemaphoreType.DMA,
        ],
    )
    def copy_kernel(table_hbm, out_hbm, buf, ld_sem, st_sem):
        wid = lax.axis_index("s") * _NC + lax.axis_index("c")
        base = wid * rows_per_w

        def load(c):
            return pltpu.make_async_copy(
                table_hbm.at[pl.ds(base + c * _CHUNK, _CHUNK)],
                buf.at[c % _NBUF],
                ld_sem,
            )

        def store(c):
            return pltpu.make_async_copy(
                buf.at[c % _NBUF],
                out_hbm.at[pl.ds(base + c * _CHUNK, _CHUNK)],
                st_sem,
            )

        for c in range(min(_NBUF, n_chunks)):
            load(c).start()
        for c in range(n_chunks):
            load(c).wait()
            store(c).start()
            nxt = c + _NBUF
            if nxt < n_chunks:
                store(c).wait()
                load(nxt).start()
        for c in range(max(n_chunks - _NBUF, 0), n_chunks):
            store(c).wait()

    return copy_kernel


def kernel(x, position_embeddings):
    n_rows = x.shape[1]
    dim = position_embeddings.shape[1]
    return _make_copy(n_rows, dim)(position_embeddings)


# speedup vs baseline: 24.7417x; 1.0009x over previous
import functools

import jax
import jax.numpy as jnp
from jax import lax
from jax.experimental import pallas as pl
from jax.experimental.pallas import tpu as pltpu
from jax.experimental.pallas import tpu_sc as plsc

_INFO = plsc.get_sparse_core_info()
_NC = _INFO.num_cores
_NS = _INFO.num_subcores
_NW = _NC * _NS

_CHUNK = 16
_NBUF = 6
_PRIME = 3


@functools.cache
def _make_copy(n_rows: int, dim: int):
    rows_per_w = n_rows // _NW
    n_chunks = rows_per_w // _CHUNK
    mesh = plsc.VectorSubcoreMesh(core_axis_name="c", subcore_axis_name="s")

    @functools.partial(
        pl.kernel,
        out_type=jax.ShapeDtypeStruct((n_rows, dim), jnp.float32),
        mesh=mesh,
        scratch_types=[
            pltpu.VMEM((_NBUF, _CHUNK, dim), jnp.float32),
            pltpu.SemaphoreType.DMA,
            pltpu.SemaphoreType.DMA,
        ],
    )
    def copy_kernel(table_hbm, out_hbm, buf, ld_sem, st_sem):
        wid = lax.axis_index("s") * _NC + lax.axis_index("c")
        base = wid * rows_per_w

        def load(c):
            return pltpu.make_async_copy(
                table_hbm.at[pl.ds(base + c * _CHUNK, _CHUNK)],
                buf.at[c % _NBUF],
                ld_sem,
            )

        def store(c):
            return pltpu.make_async_copy(
                buf.at[c % _NBUF],
                out_hbm.at[pl.ds(base + c * _CHUNK, _CHUNK)],
                st_sem,
            )

        for c in range(min(_PRIME, n_chunks)):
            load(c).start()
        for c in range(n_chunks):
            nxt = c + _PRIME
            if nxt < n_chunks:
                old = nxt - _NBUF
                if old >= 0:
                    store(old).wait()
                load(nxt).start()
            load(c).wait()
            store(c).start()
        for c in range(max(n_chunks - _NBUF, 0), n_chunks):
            store(c).wait()

    return copy_kernel


def kernel(x, position_embeddings):
    n_rows = x.shape[1]
    dim = position_embeddings.shape[1]
    return _make_copy(n_rows, dim)(position_embeddings)


# speedup vs baseline: 41.4689x; 1.6761x over previous
import functools

import jax
import jax.numpy as jnp
from jax.experimental import pallas as pl


@functools.cache
def _make_copy(n_rows: int, dim: int):
    block = 512

    def body(in_ref, o_ref):
        o_ref[...] = in_ref[...]

    return pl.pallas_call(
        body,
        grid=(n_rows // block,),
        in_specs=[pl.BlockSpec((block, dim), lambda i: (i, 0))],
        out_specs=pl.BlockSpec((block, dim), lambda i: (i, 0)),
        out_shape=jax.ShapeDtypeStruct((n_rows, dim), jnp.float32),
    )


def kernel(x, position_embeddings):
    n_rows = x.shape[1]
    dim = position_embeddings.shape[1]
    return _make_copy(n_rows, dim)(position_embeddings)
